# Initial kernel scaffold; baseline (speedup 1.0000x reference)
import jax
import jax.numpy as jnp
from jax.experimental import pallas as pl


def kernel(lr_lv3, refsr_lv3, ref_lv1, ref_lv2, ref_lv3):
    raise NotImplementedError("write your pallas kernel here")


# speedup vs baseline: 82.6447x; 82.6447x over previous
import functools

import jax
import jax.numpy as jnp
from jax import lax
from jax.experimental import pallas as pl
from jax.experimental.pallas import tpu as pltpu
from jax.experimental.pallas import tpu_sc as plsc

_HW = 40
_L = _HW * _HW
_F = 64 * 9
_CG = 42
_NCELL = _CG * _CG


def _corr_body(q_ref, k_ref, s_ref, r_ref):
    q = q_ref[0]
    k = k_ref[0]
    q = q / jnp.maximum(jnp.sqrt(jnp.sum(q * q, axis=0, keepdims=True)), 1e-12)
    k = k / jnp.maximum(jnp.sqrt(jnp.sum(k * k, axis=0, keepdims=True)), 1e-12)
    c = lax.dot_general(q, k, (((0,), (0,)), ((), ())),
                        preferred_element_type=jnp.float32)
    hq = jnp.argmax(c, axis=1).astype(jnp.int32)
    iot = lax.broadcasted_iota(jnp.int32, (_L, _L), 1)
    cnt = jnp.sum(jnp.where(hq[:, None] == iot, 1, 0), axis=0)
    present = cnt > 0
    neg = jnp.float32(-jnp.inf)
    cm = jnp.where(present[None, :], c, neg)
    v1 = jnp.max(cm, axis=1)
    r1 = jnp.argmax(cm, axis=1).astype(jnp.int32)
    e1 = iot == r1[:, None]
    cm2 = jnp.where(e1, neg, cm)
    v2d = jnp.max(cm2, axis=1)
    r2d = jnp.argmax(cm2, axis=1).astype(jnp.int32)
    dup = cnt >= 2
    dup1 = jnp.sum(jnp.where(e1 & dup[None, :], 1, 0), axis=1) > 0
    v2 = jnp.where(dup1, v1, v2d)
    r2 = jnp.where(dup1, r1, r2d)
    s_ref[0, 0, :] = v1
    s_ref[0, 1, :] = v2
    r_ref[0, 0, :] = r1
    r_ref[0, 1, :] = r2


_N_WORK = 32
_ROWS_PER_W = 200
_CHUNK = 8
_NCHUNK = _ROWS_PER_W // _CHUNK


def _make_transfer_body(d):
    cols = d // 16

    def body(idx_hbm, t_hbm, o_hbm, idx_v, tmp9, outb, sem):
        wid = lax.axis_index("s") * 2 + lax.axis_index("c")
        bs = wid // 8
        wq = wid % 8
        ob = wid * _ROWS_PER_W

        @pl.loop(0, _NCHUNK)
        def _chunk(ci):
            blk = (bs * (8 * _NCHUNK) + wq * _NCHUNK + ci) * (9 * _CHUNK)
            pltpu.sync_copy(idx_hbm.at[pl.ds(blk, 9 * _CHUNK)], idx_v)
            copies = [
                pltpu.async_copy(
                    t_hbm.at[idx_v.at[pl.ds(k * _CHUNK, _CHUNK)]],
                    tmp9.at[k], sem)
                for k in range(9)
            ]
            for c in copies:
                c.wait()
            for r in range(_CHUNK):
                @pl.loop(0, cols, unroll=4)
                def _col(cj):
                    o = cj * 16
                    a = tmp9[0, r, pl.ds(o, 16)]
                    for k in range(1, 9):
                        a = a + tmp9[k, r, pl.ds(o, 16)]
                    outb[r, pl.ds(o, 16)] = a
            pltpu.sync_copy(outb, o_hbm.at[pl.ds(ob + ci * _CHUNK, _CHUNK)])

    return body


def _unfold3x3(xpad, b, ch):
    cols = [xpad[:, :, i:i + _HW, j:j + _HW].reshape(b, ch, _L)
            for i in range(3) for j in range(3)]
    return jnp.concatenate(cols, axis=1)


def kernel(lr_lv3, refsr_lv3, ref_lv1, ref_lv2, ref_lv3):
    B = lr_lv3.shape[0]
    f32, i32 = jnp.float32, jnp.int32

    lrp = jnp.pad(lr_lv3, ((0, 0), (0, 0), (1, 1), (1, 1)))
    rfp = jnp.pad(refsr_lv3, ((0, 0), (0, 0), (1, 1), (1, 1)))
    q0 = _unfold3x3(lrp, B, 64)
    k0 = _unfold3x3(rfp, B, 64)

    s_out, r_out = pl.pallas_call(
        _corr_body,
        grid=(B,),
        in_specs=[pl.BlockSpec((1, _F, _L), lambda b: (b, 0, 0)),
                  pl.BlockSpec((1, _F, _L), lambda b: (b, 0, 0))],
        out_specs=[pl.BlockSpec((1, 2, _L), lambda b: (b, 0, 0)),
                   pl.BlockSpec((1, 2, _L), lambda b: (b, 0, 0))],
        out_shape=[jax.ShapeDtypeStruct((B, 2, _L), f32),
                   jax.ShapeDtypeStruct((B, 2, _L), i32)],
    )(q0, k0)

    ry = r_out // _HW
    rx = r_out % _HW
    rel = ((ry + 1) * _CG + rx + 1).reshape(B, 2, _HW, _HW)
    maps = []
    for dy in (-1, 0, 1):
        for dx in (-1, 0, 1):
            off = dy * _CG + dx
            p = jnp.pad(rel, ((0, 0), (0, 0), (1, 1), (1, 1)),
                        constant_values=off)
            maps.append(p[:, :, 1 + dy:41 + dy, 1 + dx:41 + dx] - off)
    idx = jnp.stack(maps, axis=2).astype(i32)
    idx = idx + (jnp.arange(B, dtype=i32) * _NCELL)[:, None, None, None, None]
    idx = (idx.reshape(B * 2, 9, _L // _CHUNK, _CHUNK)
           .transpose(0, 2, 1, 3).reshape(B * 2 * 9 * _L))

    s9 = jnp.float32(1.0 / 9.0)
    t3 = (jnp.pad(ref_lv3, ((0, 0), (0, 0), (1, 1), (1, 1))) * s9)
    t3 = t3.transpose(0, 2, 3, 1).reshape(B * _NCELL, 256)
    t2 = (jnp.pad(ref_lv2, ((0, 0), (0, 0), (2, 2), (2, 2))) * s9)
    t2 = (t2.transpose(0, 2, 3, 1).reshape(B, _CG, 2, _CG, 2, 128)
          .transpose(0, 1, 3, 2, 4, 5).reshape(B * _NCELL, 512))
    t1 = (jnp.pad(ref_lv1, ((0, 0), (0, 0), (4, 4), (4, 4))) * s9)
    t1 = (t1.transpose(0, 2, 3, 1).reshape(B, _CG, 4, _CG, 4, 64)
          .transpose(0, 1, 3, 2, 4, 5).reshape(B * _NCELL, 1024))

    nrow = B * 2 * _L

    def transfer(table, d):
        (out,) = pl.kernel(
            _make_transfer_body(d),
            out_type=[jax.ShapeDtypeStruct((nrow, d), f32)],
            mesh=plsc.VectorSubcoreMesh(core_axis_name="c",
                                        subcore_axis_name="s"),
            scratch_types=[pltpu.VMEM((9 * _CHUNK,), i32),
                           pltpu.VMEM((9, _CHUNK, d), f32),
                           pltpu.VMEM((_CHUNK, d), f32),
                           pltpu.SemaphoreType.DMA],
        )(idx, table)
        return out

    o1 = transfer(t1, 1024)
    o2 = transfer(t2, 512)
    o3 = transfer(t3, 256)

    s = s_out.transpose(1, 0, 2).reshape(2, B, 1, _HW, _HW)
    T3 = o3.reshape(B, 2, _HW, _HW, 256).transpose(1, 0, 4, 2, 3)
    T2 = (o2.reshape(B, 2, _HW, _HW, 2, 2, 128)
          .transpose(1, 0, 6, 2, 4, 3, 5).reshape(2, B, 128, 2 * _HW, 2 * _HW))
    T1 = (o1.reshape(B, 2, _HW, _HW, 4, 4, 64)
          .transpose(1, 0, 6, 2, 4, 3, 5).reshape(2, B, 64, 4 * _HW, 4 * _HW))
    return (s, T3, T2, T1)


# speedup vs baseline: 92.7517x; 1.1223x over previous
import functools

import jax
import jax.numpy as jnp
from jax import lax
from jax.experimental import pallas as pl
from jax.experimental.pallas import tpu as pltpu
from jax.experimental.pallas import tpu_sc as plsc

_HW = 40
_L = _HW * _HW
_F = 64 * 9
_CG = 42
_NCELL = _CG * _CG


def _corr_body(q_ref, k_ref, s_ref, r_ref):
    q = q_ref[0]
    k = k_ref[0]
    q = q / jnp.maximum(jnp.sqrt(jnp.sum(q * q, axis=0, keepdims=True)), 1e-12)
    k = k / jnp.maximum(jnp.sqrt(jnp.sum(k * k, axis=0, keepdims=True)), 1e-12)
    c = lax.dot_general(q, k, (((0,), (0,)), ((), ())),
                        preferred_element_type=jnp.float32)
    hq = jnp.argmax(c, axis=1).astype(jnp.int32)
    iot = lax.broadcasted_iota(jnp.int32, (_L, _L), 1)
    cnt = jnp.sum(jnp.where(hq[:, None] == iot, 1, 0), axis=0)
    present = cnt > 0
    neg = jnp.float32(-jnp.inf)
    cm = jnp.where(present[None, :], c, neg)
    v1 = jnp.max(cm, axis=1)
    r1 = jnp.argmax(cm, axis=1).astype(jnp.int32)
    e1 = iot == r1[:, None]
    cm2 = jnp.where(e1, neg, cm)
    v2d = jnp.max(cm2, axis=1)
    r2d = jnp.argmax(cm2, axis=1).astype(jnp.int32)
    dup = cnt >= 2
    dup1 = jnp.sum(jnp.where(e1 & dup[None, :], 1, 0), axis=1) > 0
    v2 = jnp.where(dup1, v1, v2d)
    r2 = jnp.where(dup1, r1, r2d)
    s_ref[0, 0, :] = v1
    s_ref[0, 1, :] = v2
    r_ref[0, 0, :] = r1
    r_ref[0, 1, :] = r2


_N_WORK = 32
_ROWS_PER_W = 200
_CHUNK = 4
_NCHUNK = _ROWS_PER_W // _CHUNK
_IDXSTRIDE = 8
_IDXPAD = 9 * _IDXSTRIDE


def _make_transfer_body(d):
    cols = d // 16

    def body(idx_hbm, t_hbm, o_hbm, idx_a, idx_b, tmp_a, tmp_b, outb,
             sem_a, sem_b):
        wid = lax.axis_index("s") * 2 + lax.axis_index("c")
        bs = wid // 8
        wq = wid % 8
        ob = wid * _ROWS_PER_W
        blk0 = bs * (8 * _NCHUNK) + wq * _NCHUNK

        def fire(c, idxv, tmp, sem):
            pltpu.sync_copy(idx_hbm.at[pl.ds((blk0 + c) * _IDXPAD, _IDXPAD)],
                            idxv)
            for k in range(9):
                pltpu.async_copy(
                    t_hbm.at[idxv.at[pl.ds(k * _IDXSTRIDE, _CHUNK)]],
                    tmp.at[k], sem)

        def drain(idxv, tmp, sem):
            for k in range(9):
                pltpu.make_async_copy(
                    t_hbm.at[idxv.at[pl.ds(k * _IDXSTRIDE, _CHUNK)]],
                    tmp.at[k], sem).wait()

        def consume(tmp, c):
            for r in range(_CHUNK):
                @pl.loop(0, cols, unroll=4)
                def _col(cj):
                    o = cj * 16
                    a = tmp[0, r, pl.ds(o, 16)]
                    for k in range(1, 9):
                        a = a + tmp[k, r, pl.ds(o, 16)]
                    outb[r, pl.ds(o, 16)] = a
            pltpu.sync_copy(outb, o_hbm.at[pl.ds(ob + c * _CHUNK, _CHUNK)])

        fire(0, idx_a, tmp_a, sem_a)

        @pl.loop(0, _NCHUNK // 2)
        def _pair(pi):
            c = pi * 2
            fire(c + 1, idx_b, tmp_b, sem_b)
            drain(idx_a, tmp_a, sem_a)
            consume(tmp_a, c)

            @pl.when(pi < _NCHUNK // 2 - 1)
            def _():
                fire(c + 2, idx_a, tmp_a, sem_a)

            drain(idx_b, tmp_b, sem_b)
            consume(tmp_b, c + 1)

    return body


def _unfold3x3(xpad, b, ch):
    cols = [xpad[:, :, i:i + _HW, j:j + _HW].reshape(b, ch, _L)
            for i in range(3) for j in range(3)]
    return jnp.concatenate(cols, axis=1)


def kernel(lr_lv3, refsr_lv3, ref_lv1, ref_lv2, ref_lv3):
    B = lr_lv3.shape[0]
    f32, i32 = jnp.float32, jnp.int32

    lrp = jnp.pad(lr_lv3, ((0, 0), (0, 0), (1, 1), (1, 1)))
    rfp = jnp.pad(refsr_lv3, ((0, 0), (0, 0), (1, 1), (1, 1)))
    q0 = _unfold3x3(lrp, B, 64)
    k0 = _unfold3x3(rfp, B, 64)

    s_out, r_out = pl.pallas_call(
        _corr_body,
        grid=(B,),
        in_specs=[pl.BlockSpec((1, _F, _L), lambda b: (b, 0, 0)),
                  pl.BlockSpec((1, _F, _L), lambda b: (b, 0, 0))],
        out_specs=[pl.BlockSpec((1, 2, _L), lambda b: (b, 0, 0)),
                   pl.BlockSpec((1, 2, _L), lambda b: (b, 0, 0))],
        out_shape=[jax.ShapeDtypeStruct((B, 2, _L), f32),
                   jax.ShapeDtypeStruct((B, 2, _L), i32)],
    )(q0, k0)

    ry = r_out // _HW
    rx = r_out % _HW
    rel = ((ry + 1) * _CG + rx + 1).reshape(B, 2, _HW, _HW)
    maps = []
    for dy in (-1, 0, 1):
        for dx in (-1, 0, 1):
            off = dy * _CG + dx
            p = jnp.pad(rel, ((0, 0), (0, 0), (1, 1), (1, 1)),
                        constant_values=off)
            maps.append(p[:, :, 1 + dy:41 + dy, 1 + dx:41 + dx] - off)
    idx = jnp.stack(maps, axis=2).astype(i32)
    idx = idx + (jnp.arange(B, dtype=i32) * _NCELL)[:, None, None, None, None]
    idx = (idx.reshape(B * 2, 9, _L // _CHUNK, _CHUNK)
           .transpose(0, 2, 1, 3))
    idx = jnp.pad(idx, ((0, 0), (0, 0), (0, 0), (0, _IDXSTRIDE - _CHUNK)))
    idx = idx.reshape(B * 2 * (_L // _CHUNK) * _IDXPAD)

    s9 = jnp.float32(1.0 / 9.0)
    t3 = (jnp.pad(ref_lv3, ((0, 0), (0, 0), (1, 1), (1, 1))) * s9)
    t3 = t3.transpose(0, 2, 3, 1).reshape(B * _NCELL, 256)
    t2 = (jnp.pad(ref_lv2, ((0, 0), (0, 0), (2, 2), (2, 2))) * s9)
    t2 = (t2.transpose(0, 2, 3, 1).reshape(B, _CG, 2, _CG, 2, 128)
          .transpose(0, 1, 3, 2, 4, 5).reshape(B * _NCELL, 512))
    t1 = (jnp.pad(ref_lv1, ((0, 0), (0, 0), (4, 4), (4, 4))) * s9)
    t1 = (t1.transpose(0, 2, 3, 1).reshape(B, _CG, 4, _CG, 4, 64)
          .transpose(0, 1, 3, 2, 4, 5).reshape(B * _NCELL, 1024))

    nrow = B * 2 * _L

    def transfer(table, d):
        (out,) = pl.kernel(
            _make_transfer_body(d),
            out_type=[jax.ShapeDtypeStruct((nrow, d), f32)],
            mesh=plsc.VectorSubcoreMesh(core_axis_name="c",
                                        subcore_axis_name="s"),
            scratch_types=[pltpu.VMEM((_IDXPAD,), i32),
                           pltpu.VMEM((_IDXPAD,), i32),
                           pltpu.VMEM((9, _CHUNK, d), f32),
                           pltpu.VMEM((9, _CHUNK, d), f32),
                           pltpu.VMEM((_CHUNK, d), f32),
                           pltpu.SemaphoreType.DMA,
                           pltpu.SemaphoreType.DMA],
        )(idx, table)
        return out

    o1 = transfer(t1, 1024)
    o2 = transfer(t2, 512)
    o3 = transfer(t3, 256)

    s = s_out.transpose(1, 0, 2).reshape(2, B, 1, _HW, _HW)
    T3 = o3.reshape(B, 2, _HW, _HW, 256).transpose(1, 0, 4, 2, 3)
    T2 = (o2.reshape(B, 2, _HW, _HW, 2, 2, 128)
          .transpose(1, 0, 6, 2, 4, 3, 5).reshape(2, B, 128, 2 * _HW, 2 * _HW))
    T1 = (o1.reshape(B, 2, _HW, _HW, 4, 4, 64)
          .transpose(1, 0, 6, 2, 4, 3, 5).reshape(2, B, 64, 4 * _HW, 4 * _HW))
    return (s, T3, T2, T1)
